# baseline (speedup 1.0000x reference)
import functools

import jax
import jax.numpy as jnp
from jax import lax
from jax.experimental import pallas as pl
from jax.experimental.pallas import tpu as pltpu
from jax.experimental.pallas import tpu_sc as plsc

VOCAB = 100000
EMB = 64
BATCH = 1024
XPAD = 128

NV = 4096

SETSZ = 4
NSET = 3


@functools.lru_cache(maxsize=None)
def _make_sc_gather():
  info = plsc.get_sparse_core_info()
  nc, ns, nl = info.num_cores, info.num_subcores, info.num_lanes
  nw = nc * ns
  b_per_w = BATCH // nw
  mesh = plsc.VectorSubcoreMesh(core_axis_name="c", subcore_axis_name="s")

  @functools.partial(
      pl.kernel,
      mesh=mesh,
      out_type=jax.ShapeDtypeStruct((BATCH, XPAD), jnp.float32),
      scratch_types=[
          pltpu.VMEM((BATCH,), jnp.int32),
          pltpu.VMEM((NSET * SETSZ, EMB, XPAD), jnp.float32),
          pltpu.VMEM((b_per_w, XPAD), jnp.float32),
          pltpu.SemaphoreType.DMA,
          pltpu.SemaphoreType.DMA,
          pltpu.SemaphoreType.DMA,
      ],
      compiler_params=pltpu.CompilerParams(needs_layout_passes=False),
  )
  def gather(table_hbm, idx_hbm, out_hbm, idx_v, fetch_v, rows_v, *sems):
    wid = lax.axis_index("s") * nc + lax.axis_index("c")
    base = wid * b_per_w
    pltpu.sync_copy(idx_hbm, idx_v)
    lane = lax.broadcasted_iota(jnp.int32, (nl,), 0)
    nround = b_per_w // SETSZ
    copies = [None] * nround
    lsplats = [None] * nround

    def issue(r):
      s = r % NSET
      cps, lsp = [], []
      for jw in range(SETSZ):
        j = r * SETSZ + jw
        cvec = idx_v[pl.ds(base + (j // nl) * nl, nl)]
        c = cvec[j % nl]
        col0 = pl.multiple_of((c >> 7) * XPAD, XPAD)
        cps.append(
            pltpu.async_copy(
                table_hbm.at[:, pl.ds(col0, XPAD)],
                fetch_v.at[s * SETSZ + jw], sems[s]))
        lsp.append(jnp.full((nl,), c & (XPAD - 1), jnp.int32))
      copies[r] = cps
      lsplats[r] = lsp

    def extract(r):
      s = r % NSET
      for cp in copies[r]:
        cp.wait()
      for jw in range(SETSZ):
        j = r * SETSZ + jw
        for kc in range(EMB // nl):
          vals = plsc.load_gather(
              fetch_v.at[s * SETSZ + jw],
              [kc * nl + lane, lsplats[r][jw]])
          rows_v[j, pl.ds(kc * nl, nl)] = vals

    for r in range(min(NSET, nround)):
      issue(r)
    for r in range(nround):
      extract(r)
      if r + NSET < nround:
        issue(r + NSET)
    pltpu.sync_copy(rows_v, out_hbm.at[pl.ds(base, b_per_w)])

  return gather


def _proj_kernel(wt_ref, x_ref, b_ref, o_ref):
  o_ref[...] = lax.dot_general(
      wt_ref[...].astype(jnp.bfloat16), x_ref[:, :EMB].astype(jnp.bfloat16),
      dimension_numbers=(((0,), (1,)), ((), ())),
      preferred_element_type=jnp.float32,
  ) + b_ref[...][:, None]


@jax.jit
def kernel(contexts, embedding, W, b):
  x_pad = _make_sc_gather()(embedding.T, contexts.astype(jnp.int32))
  Wt = W.T

  grid = pl.cdiv(VOCAB, NV)
  out_t = pl.pallas_call(
      _proj_kernel,
      grid=(grid,),
      in_specs=[
          pl.BlockSpec((EMB, NV), lambda i: (0, i)),
          pl.BlockSpec((BATCH, XPAD), lambda i: (0, 0)),
          pl.BlockSpec((NV,), lambda i: (i,)),
      ],
      out_specs=pl.BlockSpec((NV, BATCH), lambda i: (i, 0)),
      out_shape=jax.ShapeDtypeStruct((VOCAB, BATCH), jnp.float32),
  )(Wt, x_pad, b)
  return out_t.T


# speedup vs baseline: 1.0166x; 1.0166x over previous
import functools

import jax
import jax.numpy as jnp
from jax import lax
from jax.experimental import pallas as pl
from jax.experimental.pallas import tpu as pltpu
from jax.experimental.pallas import tpu_sc as plsc

VOCAB = 100000
EMB = 64
BATCH = 1024
XPAD = 128

NV = 4096

SETSZ = 2
NSET = 7


@functools.lru_cache(maxsize=None)
def _make_sc_gather():
  info = plsc.get_sparse_core_info()
  nc, ns, nl = info.num_cores, info.num_subcores, info.num_lanes
  nw = nc * ns
  b_per_w = BATCH // nw
  mesh = plsc.VectorSubcoreMesh(core_axis_name="c", subcore_axis_name="s")

  @functools.partial(
      pl.kernel,
      mesh=mesh,
      out_type=jax.ShapeDtypeStruct((BATCH, XPAD), jnp.float32),
      scratch_types=[
          pltpu.VMEM((BATCH,), jnp.int32),
          pltpu.VMEM((NSET * SETSZ, EMB, XPAD), jnp.float32),
          pltpu.VMEM((b_per_w, XPAD), jnp.float32),
      ] + [pltpu.SemaphoreType.DMA] * NSET,
      compiler_params=pltpu.CompilerParams(needs_layout_passes=False),
  )
  def gather(table_hbm, idx_hbm, out_hbm, idx_v, fetch_v, rows_v, *sems):
    wid = lax.axis_index("s") * nc + lax.axis_index("c")
    base = wid * b_per_w
    pltpu.sync_copy(idx_hbm, idx_v)
    lane = lax.broadcasted_iota(jnp.int32, (nl,), 0)
    nround = b_per_w // SETSZ
    copies = [None] * nround
    lsplats = [None] * nround

    def issue(r):
      s = r % NSET
      cps, lsp = [], []
      for jw in range(SETSZ):
        j = r * SETSZ + jw
        cvec = idx_v[pl.ds(base + (j // nl) * nl, nl)]
        c = cvec[j % nl]
        col0 = pl.multiple_of((c >> 7) * XPAD, XPAD)
        cps.append(
            pltpu.async_copy(
                table_hbm.at[:, pl.ds(col0, XPAD)],
                fetch_v.at[s * SETSZ + jw], sems[s]))
        lsp.append(jnp.full((nl,), c & (XPAD - 1), jnp.int32))
      copies[r] = cps
      lsplats[r] = lsp

    def extract(r):
      s = r % NSET
      for cp in copies[r]:
        cp.wait()
      for jw in range(SETSZ):
        j = r * SETSZ + jw
        for kc in range(EMB // nl):
          vals = plsc.load_gather(
              fetch_v.at[s * SETSZ + jw],
              [kc * nl + lane, lsplats[r][jw]])
          rows_v[j, pl.ds(kc * nl, nl)] = vals

    for r in range(min(NSET, nround)):
      issue(r)
    for r in range(nround):
      extract(r)
      if r + NSET < nround:
        issue(r + NSET)
    pltpu.sync_copy(rows_v, out_hbm.at[pl.ds(base, b_per_w)])

  return gather


def _proj_kernel(wt_ref, x_ref, b_ref, o_ref):
  o_ref[...] = lax.dot_general(
      wt_ref[...].astype(jnp.bfloat16), x_ref[:, :EMB].astype(jnp.bfloat16),
      dimension_numbers=(((0,), (1,)), ((), ())),
      preferred_element_type=jnp.float32,
  ) + b_ref[...][:, None]


@jax.jit
def kernel(contexts, embedding, W, b):
  x_pad = _make_sc_gather()(embedding.T, contexts.astype(jnp.int32))
  Wt = W.T

  grid = pl.cdiv(VOCAB, NV)
  out_t = pl.pallas_call(
      _proj_kernel,
      grid=(grid,),
      in_specs=[
          pl.BlockSpec((EMB, NV), lambda i: (0, i)),
          pl.BlockSpec((BATCH, XPAD), lambda i: (0, 0)),
          pl.BlockSpec((NV,), lambda i: (i,)),
      ],
      out_specs=pl.BlockSpec((NV, BATCH), lambda i: (i, 0)),
      out_shape=jax.ShapeDtypeStruct((VOCAB, BATCH), jnp.float32),
  )(Wt, x_pad, b)
  return out_t.T
